# Initial kernel scaffold; baseline (speedup 1.0000x reference)
import jax
import jax.numpy as jnp
from jax.experimental import pallas as pl


def kernel(feat, feat_0, edge_index, weight1, bias):
    raise NotImplementedError("write your pallas kernel here")


# speedup vs baseline: 4.2861x; 4.2861x over previous
import functools
import math

import jax
import jax.numpy as jnp
from jax import lax
from jax.experimental import pallas as pl
from jax.experimental.pallas import tpu as pltpu
from jax.experimental.pallas import tpu_sc as plsc

_N = 10000
_E = 320000
_D = 128
_ALPHA = 0.1
_BETA = math.log(1.0 / 2.0 + 1.0)

_NC = 2
_NS = 16
_NW = _NC * _NS
_EPT = _E // _NW
_CH = 80
_NCHUNK = _EPT // _CH
_NP = 10240
_RPT = _NP // _NS
_ZR = 128

_mesh = plsc.VectorSubcoreMesh(core_axis_name="c", subcore_axis_name="s")


def _deg_body(dst_hbm, degp_hbm, shared, idx_v, ones_v, zer_v):
    cid = lax.axis_index("c")
    sid = lax.axis_index("s")

    @pl.loop(0, _ZR)
    def _(i):
        @pl.loop(0, _D // 16)
        def _(j):
            zer_v[i, pl.ds(j * 16, 16)] = jnp.zeros((16,), jnp.float32)

    @pl.loop(0, _CH)
    def _(i):
        @pl.loop(0, _D // 16)
        def _(j):
            ones_v[i, pl.ds(j * 16, 16)] = jnp.ones((16,), jnp.float32)

    rbase = sid * _RPT

    @pl.loop(0, _RPT // _ZR)
    def _(j):
        pltpu.sync_copy(zer_v, shared.at[pl.ds(rbase + j * _ZR, _ZR)])

    plsc.subcore_barrier()

    ebase = (sid * _NC + cid) * _EPT

    @pl.loop(0, _NCHUNK)
    def _(t):
        pltpu.sync_copy(dst_hbm.at[pl.ds(ebase + t * _CH, _CH)], idx_v)
        pltpu.sync_copy(ones_v, shared.at[idx_v], add=True)

    plsc.subcore_barrier()
    pltpu.sync_copy(shared.at[pl.ds(rbase, _RPT)],
                    degp_hbm.at[cid].at[pl.ds(rbase, _RPT)])


def _agg_body(g_hbm, src_hbm, dst_hbm, aggp_hbm,
              shared, sidx_v, didx_v, rows_v, zer_v):
    cid = lax.axis_index("c")
    sid = lax.axis_index("s")

    @pl.loop(0, _ZR)
    def _(i):
        @pl.loop(0, _D // 16)
        def _(j):
            zer_v[i, pl.ds(j * 16, 16)] = jnp.zeros((16,), jnp.float32)

    rbase = sid * _RPT

    @pl.loop(0, _RPT // _ZR)
    def _(j):
        pltpu.sync_copy(zer_v, shared.at[pl.ds(rbase + j * _ZR, _ZR)])

    plsc.subcore_barrier()

    ebase = (sid * _NC + cid) * _EPT

    @pl.loop(0, _NCHUNK)
    def _(t):
        pltpu.sync_copy(src_hbm.at[pl.ds(ebase + t * _CH, _CH)], sidx_v)
        pltpu.sync_copy(dst_hbm.at[pl.ds(ebase + t * _CH, _CH)], didx_v)
        pltpu.sync_copy(g_hbm.at[sidx_v], rows_v)
        pltpu.sync_copy(rows_v, shared.at[didx_v], add=True)

    plsc.subcore_barrier()
    pltpu.sync_copy(shared.at[pl.ds(rbase, _RPT)],
                    aggp_hbm.at[cid].at[pl.ds(rbase, _RPT)])


def _g_body(degp_ref, feat_ref, g_ref):
    deg = degp_ref[0, 0:_N, 0:1] + degp_ref[1, 0:_N, 0:1]
    norm = lax.rsqrt(jnp.maximum(deg, 1.0))
    g_ref[...] = feat_ref[...] * norm


def _final_body(aggp_ref, degp_ref, f0_ref, w_ref, b_ref, out_ref):
    deg = degp_ref[0, 0:_N, 0:1] + degp_ref[1, 0:_N, 0:1]
    norm = lax.rsqrt(jnp.maximum(deg, 1.0))
    agg = aggp_ref[0, 0:_N, :] + aggp_ref[1, 0:_N, :]
    rst0 = agg * norm * (1.0 - _ALPHA) + _ALPHA * f0_ref[...]
    rst = (1.0 - _BETA) * rst0 + _BETA * jnp.dot(
        rst0, w_ref[...], preferred_element_type=jnp.float32)
    out_ref[...] = rst + b_ref[...]


_deg_call = functools.partial(
    pl.kernel,
    out_type=jax.ShapeDtypeStruct((_NC, _NP, _D), jnp.float32),
    mesh=_mesh,
    scratch_types=[
        pltpu.VMEM_SHARED((_NP, _D), jnp.float32),
        pltpu.VMEM((_CH,), jnp.int32),
        pltpu.VMEM((_CH, _D), jnp.float32),
        pltpu.VMEM((_ZR, _D), jnp.float32),
    ],
)(_deg_body)


_agg_call = functools.partial(
    pl.kernel,
    out_type=jax.ShapeDtypeStruct((_NC, _NP, _D), jnp.float32),
    mesh=_mesh,
    scratch_types=[
        pltpu.VMEM_SHARED((_NP, _D), jnp.float32),
        pltpu.VMEM((_CH,), jnp.int32),
        pltpu.VMEM((_CH,), jnp.int32),
        pltpu.VMEM((_CH, _D), jnp.float32),
        pltpu.VMEM((_ZR, _D), jnp.float32),
    ],
)(_agg_body)


def kernel(feat, feat_0, edge_index, weight1, bias):
    src = edge_index[0]
    dst = edge_index[1]

    degp = _deg_call(dst)

    g = pl.pallas_call(
        _g_body,
        out_shape=jax.ShapeDtypeStruct((_N, _D), jnp.float32),
    )(degp, feat)

    aggp = _agg_call(g, src, dst)

    out = pl.pallas_call(
        _final_body,
        out_shape=jax.ShapeDtypeStruct((_N, _D), jnp.float32),
    )(aggp, degp, feat_0, weight1, bias.reshape(1, _D))
    return out


# speedup vs baseline: 12.0984x; 2.8227x over previous
import functools
import math

import jax
import jax.numpy as jnp
from jax import lax
from jax.experimental import pallas as pl
from jax.experimental.pallas import tpu as pltpu
from jax.experimental.pallas import tpu_sc as plsc

_N = 10000
_E = 320000
_D = 128
_ALPHA = 0.1
_BETA = math.log(1.0 / 2.0 + 1.0)

_NC = 2
_NS = 16
_NW = _NC * _NS
_CH = 40
_NCH = _E // _CH
_CPT = _NCH // _NW
_NB = 5
_NP = 10240
_RPT = _NP // _NS
_ZRD = 128
_ZRA = 16

_mesh = plsc.VectorSubcoreMesh(core_axis_name="c", subcore_axis_name="s")
_sc_params = pltpu.CompilerParams(use_tc_tiling_on_sc=False)


def _deg_body(dst2_hbm, degp_hbm, shared, idx_v, ones_v, zer_v,
              s0, s1, s2, s3, s4):
    cid = lax.axis_index("c")
    sid = lax.axis_index("s")
    ssem = (s0, s1, s2, s3, s4)

    @pl.loop(0, _ZRD)
    def _(i):
        zer_v[i] = jnp.zeros((16,), jnp.float32)

    @pl.loop(0, _CH)
    def _(i):
        ones_v[i] = jnp.ones((16,), jnp.float32)

    rbase = sid * _RPT

    @pl.loop(0, _RPT // _ZRD)
    def _(j):
        pltpu.sync_copy(zer_v, shared.at[pl.ds(rbase + j * _ZRD, _ZRD)])

    plsc.subcore_barrier()

    crow = (sid * _NC + cid) * _CPT
    pltpu.sync_copy(dst2_hbm.at[pl.ds(crow, _CPT)], idx_v)

    @pl.loop(0, _CPT // _NB)
    def _(g):
        for b in range(_NB):
            @pl.when(g > 0)
            def _():
                pltpu.make_async_copy(
                    ones_v, shared.at[idx_v.at[0]], ssem[b]).wait()

            pltpu.async_copy(
                ones_v, shared.at[idx_v.at[g * _NB + b]], ssem[b], add=True)

    for b in range(_NB):
        pltpu.make_async_copy(ones_v, shared.at[idx_v.at[0]], ssem[b]).wait()

    plsc.subcore_barrier()
    pltpu.sync_copy(shared.at[pl.ds(rbase, _RPT)],
                    degp_hbm.at[cid].at[pl.ds(rbase, _RPT)])


def _agg_body(g_hbm, src2_hbm, dst2_hbm, aggp_hbm,
              shared, sidx_v, didx_v, r0, r1, r2, r3, r4, zer_v,
              g0, g1, g2, g3, g4, t0, t1, t2, t3, t4):
    cid = lax.axis_index("c")
    sid = lax.axis_index("s")
    rows = (r0, r1, r2, r3, r4)
    gsem = (g0, g1, g2, g3, g4)
    ssem = (t0, t1, t2, t3, t4)

    @pl.loop(0, _ZRA)
    def _(i):
        @pl.loop(0, _D // 16)
        def _(j):
            zer_v[i, pl.ds(j * 16, 16)] = jnp.zeros((16,), jnp.float32)

    rbase = sid * _RPT

    @pl.loop(0, _RPT // _ZRA)
    def _(j):
        pltpu.sync_copy(zer_v, shared.at[pl.ds(rbase + j * _ZRA, _ZRA)])

    plsc.subcore_barrier()

    crow = (sid * _NC + cid) * _CPT
    pltpu.sync_copy(src2_hbm.at[pl.ds(crow, _CPT)], sidx_v)
    pltpu.sync_copy(dst2_hbm.at[pl.ds(crow, _CPT)], didx_v)

    def issue_gather(t, b):
        pltpu.async_copy(g_hbm.at[sidx_v.at[t]], rows[b], gsem[b])

    def wait_gather(t, b):
        pltpu.make_async_copy(g_hbm.at[sidx_v.at[t]], rows[b], gsem[b]).wait()

    def issue_scatter(t, b):
        pltpu.async_copy(rows[b], shared.at[didx_v.at[t]], ssem[b], add=True)

    def wait_scatter(b):
        pltpu.make_async_copy(rows[b], shared.at[didx_v.at[0]], ssem[b]).wait()

    for b in range(_NB - 1):
        issue_gather(b, b)

    wait_gather(0, 0)
    issue_scatter(0, 0)
    issue_gather(_NB - 1, _NB - 1)
    for u in range(1, _NB):
        b = u % _NB
        wait_gather(u, b)
        issue_scatter(u, b)
        pb = (b + _NB - 1) % _NB
        wait_scatter(pb)
        issue_gather(u + _NB - 1, pb)

    @pl.loop(0, (_CPT - _NB) // _NB)
    def _(gg):
        for b in range(_NB):
            u = _NB + gg * _NB + b
            wait_gather(u, b)
            issue_scatter(u, b)
            pb = (b + _NB - 1) % _NB
            wait_scatter(pb)

            @pl.when(u + _NB - 1 <= _CPT - 1)
            def _():
                issue_gather(u + _NB - 1, pb)

    wait_scatter((_CPT - 1) % _NB)

    plsc.subcore_barrier()
    pltpu.sync_copy(shared.at[pl.ds(rbase, _RPT)],
                    aggp_hbm.at[cid].at[pl.ds(rbase, _RPT)])


def _g_body(degp_ref, feat_ref, g_ref):
    deg = degp_ref[0, 0:_N, 0:1] + degp_ref[1, 0:_N, 0:1]
    norm = lax.rsqrt(jnp.maximum(deg, 1.0))
    g_ref[...] = feat_ref[...] * norm


def _final_body(aggp_ref, degp_ref, f0_ref, w_ref, b_ref, out_ref):
    deg = degp_ref[0, 0:_N, 0:1] + degp_ref[1, 0:_N, 0:1]
    norm = lax.rsqrt(jnp.maximum(deg, 1.0))
    agg = aggp_ref[0, 0:_N, :] + aggp_ref[1, 0:_N, :]
    rst0 = agg * norm * (1.0 - _ALPHA) + _ALPHA * f0_ref[...]
    rst = (1.0 - _BETA) * rst0 + _BETA * jnp.dot(
        rst0, w_ref[...], preferred_element_type=jnp.float32)
    out_ref[...] = rst + b_ref[...]


_deg_call = functools.partial(
    pl.kernel,
    out_type=jax.ShapeDtypeStruct((_NC, _NP, 16), jnp.float32),
    mesh=_mesh,
    compiler_params=_sc_params,
    scratch_types=[
        pltpu.VMEM_SHARED((_NP, 16), jnp.float32),
        pltpu.VMEM((_CPT, _CH), jnp.int32),
        pltpu.VMEM((_CH, 16), jnp.float32),
        pltpu.VMEM((_ZRD, 16), jnp.float32),
    ] + [pltpu.SemaphoreType.DMA] * _NB,
)(_deg_body)


_agg_call = functools.partial(
    pl.kernel,
    out_type=jax.ShapeDtypeStruct((_NC, _NP, _D), jnp.float32),
    mesh=_mesh,
    compiler_params=_sc_params,
    scratch_types=[
        pltpu.VMEM_SHARED((_NP, _D), jnp.float32),
        pltpu.VMEM((_CPT, _CH), jnp.int32),
        pltpu.VMEM((_CPT, _CH), jnp.int32),
    ] + [pltpu.VMEM((_CH, _D), jnp.float32)] * _NB + [
        pltpu.VMEM((_ZRA, _D), jnp.float32),
    ] + [pltpu.SemaphoreType.DMA] * (2 * _NB),
)(_agg_body)


def kernel(feat, feat_0, edge_index, weight1, bias):
    src2 = edge_index[0].reshape(_NCH, _CH)
    dst2 = edge_index[1].reshape(_NCH, _CH)

    degp = _deg_call(dst2)

    g = pl.pallas_call(
        _g_body,
        out_shape=jax.ShapeDtypeStruct((_N, _D), jnp.float32),
    )(degp, feat)

    aggp = _agg_call(g, src2, dst2)

    out = pl.pallas_call(
        _final_body,
        out_shape=jax.ShapeDtypeStruct((_N, _D), jnp.float32),
    )(aggp, degp, feat_0, weight1, bias.reshape(1, _D))
    return out
